# baseline (speedup 1.0000x reference)
import jax
import jax.numpy as jnp
from jax.experimental import pallas as pl

_C_BLK = 2048


def _sim_kernel(f_ref, p_ref, o_ref):
    f = f_ref[...]
    norm = jnp.sqrt(jnp.sum(f * f, axis=1, keepdims=True))
    fn = f / jnp.maximum(norm, 1e-12)
    o_ref[...] = jax.lax.dot_general(
        fn, p_ref[...], (((1,), (1,)), ((), ())),
        preferred_element_type=jnp.float32)


def kernel(feats, prototypes):
    batch, emb = feats.shape
    n_classes = prototypes.shape[0]
    return pl.pallas_call(
        _sim_kernel,
        grid=(pl.cdiv(n_classes, _C_BLK),),
        in_specs=[
            pl.BlockSpec((batch, emb), lambda i: (0, 0)),
            pl.BlockSpec((_C_BLK, emb), lambda i: (i, 0)),
        ],
        out_specs=pl.BlockSpec((batch, _C_BLK), lambda i: (0, i)),
        out_shape=jax.ShapeDtypeStruct((batch, n_classes), jnp.float32),
    )(feats, prototypes)


# speedup vs baseline: 1.0927x; 1.0927x over previous
import jax
import jax.numpy as jnp
from jax.experimental import pallas as pl

_B_BLK = 32


def _sim_kernel(f_ref, pt_ref, o_ref):
    f = f_ref[...]
    norm = jnp.sqrt(jnp.sum(f * f, axis=1, keepdims=True))
    fn = f / jnp.maximum(norm, 1e-12)
    o_ref[...] = jnp.dot(fn, pt_ref[...], preferred_element_type=jnp.float32)


def kernel(feats, prototypes):
    batch, emb = feats.shape
    n_classes = prototypes.shape[0]
    pt = prototypes.T
    return pl.pallas_call(
        _sim_kernel,
        grid=(pl.cdiv(batch, _B_BLK),),
        in_specs=[
            pl.BlockSpec((_B_BLK, emb), lambda i: (i, 0)),
            pl.BlockSpec((emb, n_classes), lambda i: (0, 0)),
        ],
        out_specs=pl.BlockSpec((_B_BLK, n_classes), lambda i: (i, 0)),
        out_shape=jax.ShapeDtypeStruct((batch, n_classes), jnp.float32),
    )(feats, pt)


# speedup vs baseline: 1.1218x; 1.0266x over previous
import jax
import jax.numpy as jnp
from jax.experimental import pallas as pl
from jax.experimental.pallas import tpu as pltpu

_B_BLK = 16
_NBUF = 4


def _sim_kernel(f_ref, pt_ref, o_hbm, buf, sems):
    i = pl.program_id(0)
    n = pl.num_programs(0)
    slot = jax.lax.rem(i, _NBUF)

    @pl.when(i >= _NBUF)
    def _wait_prev():
        pltpu.make_async_copy(
            buf.at[slot],
            o_hbm.at[pl.ds((i - _NBUF) * _B_BLK, _B_BLK), :],
            sems.at[slot],
        ).wait()

    f = f_ref[...]
    norm = jnp.sqrt(jnp.sum(f * f, axis=1, keepdims=True))
    fn = f / jnp.maximum(norm, 1e-12)
    buf[slot] = jnp.dot(fn, pt_ref[...], preferred_element_type=jnp.float32)

    pltpu.make_async_copy(
        buf.at[slot],
        o_hbm.at[pl.ds(i * _B_BLK, _B_BLK), :],
        sems.at[slot],
    ).start()

    @pl.when(i == n - 1)
    def _drain():
        for k in range(min(_NBUF, n)):
            s = n - 1 - k
            pltpu.make_async_copy(
                buf.at[s % _NBUF],
                o_hbm.at[pl.ds(s * _B_BLK, _B_BLK), :],
                sems.at[s % _NBUF],
            ).wait()


def kernel(feats, prototypes):
    batch, emb = feats.shape
    n_classes = prototypes.shape[0]
    pt = prototypes.T
    return pl.pallas_call(
        _sim_kernel,
        grid=(batch // _B_BLK,),
        in_specs=[
            pl.BlockSpec((_B_BLK, emb), lambda i: (i, 0)),
            pl.BlockSpec((emb, n_classes), lambda i: (0, 0)),
        ],
        out_specs=pl.BlockSpec(memory_space=pl.MemorySpace.ANY),
        out_shape=jax.ShapeDtypeStruct((batch, n_classes), jnp.float32),
        scratch_shapes=[
            pltpu.VMEM((_NBUF, _B_BLK, n_classes), jnp.float32),
            pltpu.SemaphoreType.DMA((_NBUF,)),
        ],
    )(feats, pt)
